# baseline (speedup 1.0000x reference)
import functools

import jax
import jax.numpy as jnp
from jax import lax
from jax.experimental import pallas as pl
from jax.experimental.pallas import tpu as pltpu
from jax.experimental.pallas import tpu_sc as plsc

_V = 1000000
_B = 16384
_D = 64
_S = 16

_info = plsc.get_sparse_core_info()
_NC = _info.num_cores
_NW = _info.num_cores * _info.num_subcores
_BPW = _B // _NW
_NG = _V // 8
_WG = 16
_NWIN = (_NG // _NW + 1 + _WG - 1) // _WG
_ACC = 256
_DUMP = _B

_mesh = plsc.VectorSubcoreMesh(core_axis_name="c", subcore_axis_name="s")
_params = pltpu.CompilerParams(needs_layout_passes=False,
                               skip_device_barrier=True)


@functools.partial(
    pl.kernel,
    mesh=_mesh,
    compiler_params=_params,
    out_type=jax.ShapeDtypeStruct((_B + _ACC, 128), jnp.float32),
    scratch_types=[
        pltpu.VMEM((_B,), jnp.int32),
        pltpu.VMEM((_B,), jnp.int32),
        pltpu.VMEM((_B,), jnp.int32),
        pltpu.VMEM((_WG, 8, _D), jnp.float32),
        pltpu.VMEM((_WG, 8, _D), jnp.float32),
        pltpu.VMEM((_ACC, 128), jnp.float32),
        pltpu.VMEM((_ACC,), jnp.int32),
        pltpu.SemaphoreType.DMA,
        pltpu.SemaphoreType.DMA,
        pltpu.SemaphoreType.DMA,
    ],
)
def _scan_kernel(idx_hbm, table_hbm, scratch_hbm,
                 idx_v, hval_v, hpos_v, win0_v, win1_v, acc_v, apos_v,
                 sem0, sem1, fsem):
    wid = lax.axis_index("s") * _NC + lax.axis_index("c")
    table_g = table_hbm.reshape(_NG, 8, _D)
    glo = wid * _NG // _NW
    ghi = (wid + 1) * _NG // _NW
    gsize = ghi - glo
    shard_lo = glo * 8
    shard_hi = ghi * 8
    pltpu.sync_copy(idx_hbm.at[:], idx_v)

    lanes = lax.iota(jnp.int32, 16)
    big = jnp.full((16,), 0x7FFFFFF, dtype=jnp.int32)

    def prefilter(i, cnt):
        v = idx_v[pl.ds(i * 16, 16)]
        m = jnp.logical_and(v >= shard_lo, v < shard_hi)
        c = jnp.max(plsc.all_reduce_population_count(m))

        @pl.when(c > 0)
        def _():
            plsc.store_compressed(hval_v.at[pl.ds(cnt, 16)], v - shard_lo, mask=m)
            plsc.store_compressed(hpos_v.at[pl.ds(cnt, 16)], lanes + i * 16, mask=m)

        return cnt + c

    def initbig(i, carry):
        hval_v[pl.ds(i * 16, 16)] = big
        return carry

    lax.fori_loop(0, _B // 16, initbig, 0, unroll=8)
    hcnt = lax.fori_loop(0, _B // 16, prefilter, 0, unroll=1)
    nblk = lax.div(hcnt + 15, 16)

    def adump(i, carry):
        apos_v[pl.ds(i * 16, 16)] = lanes + _DUMP + i * 16
        return carry

    lax.fori_loop(0, _ACC // 16, adump, 0, unroll=4)

    def wstart(w):
        return jnp.minimum(w * _WG, gsize - _WG)

    def wfetch(w, wv, wsem):
        pltpu.async_copy(table_g.at[pl.ds(glo + wstart(w), _WG)], wv, wsem)

    wfetch(0, win0_v, sem0)

    def process(w, wv, wsem, acnt):
        wlo = wstart(w) * 8
        pltpu.make_async_copy(table_g.at[pl.ds(0, _WG)], wv, wsem).wait()

        def hquad(q, acnt2):
            @pl.when(acnt2 > _ACC - 64)
            def _():
                pltpu.async_copy(acc_v, scratch_hbm.at[apos_v], fsem).wait()
                lax.fori_loop(0, _ACC // 16, adump, 0, unroll=4)

            acnt2 = jnp.where(acnt2 > _ACC - 64, 0, acnt2)
            for k in range(4):
                b = q * 4 + k
                v = hval_v[pl.ds(b * 16, 16)]
                m = jnp.logical_and(v >= wlo, v < wlo + _WG * 8)
                c = jnp.max(plsc.all_reduce_population_count(m))

                @pl.when(c > 0)
                def _(m=m, v=v, b=b, base=acnt2):
                    local = jnp.where(m, v - wlo, 0)
                    lg = lax.shift_right_logical(local, 3)
                    lr = lax.bitwise_and(local, 7)
                    crow = plsc.cumsum(jnp.where(m, 1, 0)) - 1 + base
                    pos = hpos_v[pl.ds(b * 16, 16)]
                    plsc.store_scatter(apos_v, [crow], pos, mask=m)

                    def col(j, carry3):
                        g = plsc.load_gather(wv, [lg, lr, lanes * 0 + j], mask=m)
                        plsc.store_scatter(acc_v, [crow, lanes * 0 + j], g, mask=m)
                        return carry3

                    lax.fori_loop(0, _D, col, 0, unroll=8)

                acnt2 = acnt2 + c
            return acnt2

        return lax.fori_loop(0, lax.div(nblk + 3, 4), hquad, acnt, unroll=1)

    def wpair(wp, acnt):
        w0 = wp * 2
        wfetch(w0 + 1, win1_v, sem1)
        acnt = process(w0, win0_v, sem0, acnt)

        @pl.when(w0 + 2 < _NWIN)
        def _():
            wfetch(w0 + 2, win0_v, sem0)

        acnt = process(w0 + 1, win1_v, sem1, acnt)
        return acnt

    acnt = lax.fori_loop(0, (_NWIN + 1) // 2, wpair, 0, unroll=1)

    @pl.when(acnt > 0)
    def _():
        pltpu.async_copy(acc_v, scratch_hbm.at[apos_v], fsem).wait()


@functools.partial(
    pl.kernel,
    mesh=_mesh,
    compiler_params=_params,
    out_type=jax.ShapeDtypeStruct((_B, _D), jnp.float32),
    scratch_types=[
        pltpu.VMEM((_BPW // 2, 128), jnp.float32),
        pltpu.VMEM((_BPW // 2, _D), jnp.float32),
        pltpu.VMEM((_S,), jnp.float32),
    ],
)
def _fix_kernel(scratch_hbm, shared_hbm, out_hbm, rows128_v, rows64_v, sh_v):
    wid = lax.axis_index("s") * _NC + lax.axis_index("c")
    base = wid * _BPW
    pltpu.sync_copy(shared_hbm.at[0], sh_v)
    sval = sh_v[...]
    half = _BPW // 2
    for h in range(2):
        pltpu.sync_copy(scratch_hbm.at[pl.ds(base + h * half, half)], rows128_v)

        def body(i, carry):
            rows64_v[i, pl.ds(0, _S)] = sval
            for k in range(1, _D // 16):
                rows64_v[i, pl.ds(k * 16, 16)] = rows128_v[i, pl.ds(k * 16, 16)]
            return carry

        lax.fori_loop(0, half, body, 0, unroll=4)
        pltpu.sync_copy(rows64_v, out_hbm.at[pl.ds(base + h * half, half)])


def kernel(X, W, shared_embed):
    scratch = _scan_kernel(X.astype(jnp.int32), W)
    return _fix_kernel(scratch, shared_embed)


# speedup vs baseline: 2.1430x; 2.1430x over previous
import functools

import jax
import jax.numpy as jnp
from jax import lax
from jax.experimental import pallas as pl
from jax.experimental.pallas import tpu as pltpu
from jax.experimental.pallas import tpu_sc as plsc

_B = 16384
_D = 64
_S = 16

_info = plsc.get_sparse_core_info()
_NC = _info.num_cores
_NW = _info.num_cores * _info.num_subcores
_BPW = _B // _NW

_mesh = plsc.VectorSubcoreMesh(core_axis_name="c", subcore_axis_name="s")


@functools.partial(
    pl.kernel,
    mesh=_mesh,
    compiler_params=pltpu.CompilerParams(needs_layout_passes=False),
    out_type=jax.ShapeDtypeStruct((_B, _D), jnp.float32),
    scratch_types=[
        pltpu.VMEM((_BPW,), jnp.int32),
        pltpu.VMEM((_BPW, _D), jnp.float32),
        pltpu.VMEM((_S,), jnp.float32),
        pltpu.SemaphoreType.DMA,
    ],
)
def _emb_kernel(idx_hbm, table_hbm, shared_hbm, out_hbm,
                idx_v, rows_v, sh_v, sem):
    wid = lax.axis_index("s") * _NC + lax.axis_index("c")
    base = wid * _BPW
    pltpu.sync_copy(idx_hbm.at[pl.ds(base, _BPW)], idx_v)
    pltpu.sync_copy(shared_hbm.at[0], sh_v)

    lanes = lax.iota(jnp.int32, 16)

    def fire(blk, carry):
        vec = idx_v[pl.ds(blk * 16, 16)]
        for l in range(16):
            r = jnp.sum(jnp.where(lanes == l, vec, 0))
            pltpu.async_copy(table_hbm.at[r], rows_v.at[blk * 16 + l], sem)
        return carry

    lax.fori_loop(0, _BPW // 16, fire, 0, unroll=1)

    def drain(i, carry):
        pltpu.make_async_copy(table_hbm.at[0], rows_v.at[i], sem).wait()
        return carry

    lax.fori_loop(0, _BPW, drain, 0, unroll=8)

    sval = sh_v[...]

    def body(i, carry):
        rows_v[i, pl.ds(0, _S)] = sval
        return carry

    lax.fori_loop(0, _BPW, body, 0, unroll=8)
    pltpu.sync_copy(rows_v, out_hbm.at[pl.ds(base, _BPW)])


def kernel(X, W, shared_embed):
    return _emb_kernel(X.astype(jnp.int32), W, shared_embed)
